# baseline (speedup 1.0000x reference)
import functools

import jax
import jax.numpy as jnp
from jax import lax
from jax.experimental import pallas as pl
from jax.experimental.pallas import tpu as pltpu
from jax.experimental.pallas import tpu_sc as plsc

NUM_FEATS = 26
VOCAB = 100000
EMBED = 64
BATCH = 4096

_NC = 2
_NS = 16
_NW = _NC * _NS
_BPW = BATCH // _NW


def _body(xs_hbm, *refs):
    ws = refs[:NUM_FEATS]
    out_hbm = refs[NUM_FEATS]
    idx_v, rows_v, sem = refs[NUM_FEATS + 1:]

    wid = lax.axis_index("s") * _NC + lax.axis_index("c")
    base = wid * _BPW

    pltpu.sync_copy(xs_hbm.at[:, pl.ds(base, _BPW)], idx_v)

    for f in range(NUM_FEATS):
        pltpu.async_copy(ws[f].at[idx_v.at[f]], rows_v, sem).wait()
        pltpu.sync_copy(rows_v, out_hbm.at[pl.ds(base, _BPW), f])


@jax.jit
def _run(xs, *ws):
    mesh = plsc.VectorSubcoreMesh(core_axis_name="c", subcore_axis_name="s")
    return pl.kernel(
        _body,
        out_type=jax.ShapeDtypeStruct((BATCH, NUM_FEATS, EMBED), jnp.float32),
        mesh=mesh,
        scratch_types=[
            pltpu.VMEM((NUM_FEATS, _BPW), jnp.int32),
            pltpu.VMEM((_BPW, EMBED), jnp.float32),
            pltpu.SemaphoreType.DMA,
        ],
        compiler_params=pltpu.CompilerParams(use_tc_tiling_on_sc=False),
    )(xs, *ws)


def kernel(X_0, X_1, X_2, X_3, X_4, X_5, X_6, X_7, X_8, X_9, X_10, X_11, X_12, X_13, X_14, X_15, X_16, X_17, X_18, X_19, X_20, X_21, X_22, X_23, X_24, X_25, W_0, W_1, W_2, W_3, W_4, W_5, W_6, W_7, W_8, W_9, W_10, W_11, W_12, W_13, W_14, W_15, W_16, W_17, W_18, W_19, W_20, W_21, W_22, W_23, W_24, W_25):
    xs = jnp.stack([X_0, X_1, X_2, X_3, X_4, X_5, X_6, X_7, X_8, X_9,
                    X_10, X_11, X_12, X_13, X_14, X_15, X_16, X_17, X_18,
                    X_19, X_20, X_21, X_22, X_23, X_24, X_25]).astype(jnp.int32)
    ws = (W_0, W_1, W_2, W_3, W_4, W_5, W_6, W_7, W_8, W_9, W_10, W_11,
          W_12, W_13, W_14, W_15, W_16, W_17, W_18, W_19, W_20, W_21,
          W_22, W_23, W_24, W_25)
    return _run(xs, *ws)


# speedup vs baseline: 1.0125x; 1.0125x over previous
import functools

import jax
import jax.numpy as jnp
from jax import lax
from jax.experimental import pallas as pl
from jax.experimental.pallas import tpu as pltpu
from jax.experimental.pallas import tpu_sc as plsc

NUM_FEATS = 26
VOCAB = 100000
EMBED = 64
BATCH = 4096

_NC = 2
_NS = 16
_NW = _NC * _NS
_BPW = BATCH // _NW


_NB = 4


def _body(xs_hbm, *refs):
    ws = refs[:NUM_FEATS]
    out_hbm = refs[NUM_FEATS]
    idx_v = refs[NUM_FEATS + 1]
    rows = refs[NUM_FEATS + 2:NUM_FEATS + 2 + _NB]
    gsems = refs[NUM_FEATS + 2 + _NB:NUM_FEATS + 2 + 2 * _NB]
    wsems = refs[NUM_FEATS + 2 + 2 * _NB:]

    wid = lax.axis_index("s") * _NC + lax.axis_index("c")
    base = wid * _BPW

    pltpu.sync_copy(xs_hbm.at[:, pl.ds(base, _BPW)], idx_v)

    gd = [None] * NUM_FEATS
    wd = [None] * NUM_FEATS
    for f in range(_NB):
        gd[f] = pltpu.async_copy(ws[f].at[idx_v.at[f]], rows[f], gsems[f])
    for f in range(NUM_FEATS):
        s = f % _NB
        gd[f].wait()
        wd[f] = pltpu.async_copy(rows[s], out_hbm.at[pl.ds(base, _BPW), f],
                                 wsems[s])
        nf = f + _NB
        if nf < NUM_FEATS:
            wd[f].wait()
            gd[nf] = pltpu.async_copy(ws[nf].at[idx_v.at[nf]], rows[s],
                                      gsems[s])
    for f in range(NUM_FEATS - _NB, NUM_FEATS):
        wd[f].wait()


@jax.jit
def _run(xs, *ws):
    mesh = plsc.VectorSubcoreMesh(core_axis_name="c", subcore_axis_name="s")
    return pl.kernel(
        _body,
        out_type=jax.ShapeDtypeStruct((BATCH, NUM_FEATS, EMBED), jnp.float32),
        mesh=mesh,
        scratch_types=(
            [pltpu.VMEM((NUM_FEATS, _BPW), jnp.int32)]
            + [pltpu.VMEM((_BPW, EMBED), jnp.float32) for _ in range(_NB)]
            + [pltpu.SemaphoreType.DMA for _ in range(2 * _NB)]
        ),
        compiler_params=pltpu.CompilerParams(use_tc_tiling_on_sc=False),
    )(xs, *ws)


def kernel(X_0, X_1, X_2, X_3, X_4, X_5, X_6, X_7, X_8, X_9, X_10, X_11, X_12, X_13, X_14, X_15, X_16, X_17, X_18, X_19, X_20, X_21, X_22, X_23, X_24, X_25, W_0, W_1, W_2, W_3, W_4, W_5, W_6, W_7, W_8, W_9, W_10, W_11, W_12, W_13, W_14, W_15, W_16, W_17, W_18, W_19, W_20, W_21, W_22, W_23, W_24, W_25):
    xs = jnp.stack([X_0, X_1, X_2, X_3, X_4, X_5, X_6, X_7, X_8, X_9,
                    X_10, X_11, X_12, X_13, X_14, X_15, X_16, X_17, X_18,
                    X_19, X_20, X_21, X_22, X_23, X_24, X_25]).astype(jnp.int32)
    ws = (W_0, W_1, W_2, W_3, W_4, W_5, W_6, W_7, W_8, W_9, W_10, W_11,
          W_12, W_13, W_14, W_15, W_16, W_17, W_18, W_19, W_20, W_21,
          W_22, W_23, W_24, W_25)
    return _run(xs, *ws)


# speedup vs baseline: 1.3742x; 1.3572x over previous
import functools

import jax
import jax.numpy as jnp
from jax import lax
from jax.experimental import pallas as pl
from jax.experimental.pallas import tpu as pltpu
from jax.experimental.pallas import tpu_sc as plsc

NUM_FEATS = 26
VOCAB = 100000
EMBED = 64
BATCH = 4096

_NC = 2
_NS = 16
_NW = _NC * _NS
_BPW = BATCH // _NW


def _body(xs_hbm, *refs):
    ws = refs[:NUM_FEATS]
    out_hbm = refs[NUM_FEATS]
    ids_s, stage_v, gsem, wsem = refs[NUM_FEATS + 1:]

    wid = lax.axis_index("s") * _NC + lax.axis_index("c")
    base = wid * _BPW

    for f in range(NUM_FEATS):
        pltpu.sync_copy(xs_hbm.at[f, pl.ds(base, _BPW)], ids_s)

        def _fetch(g, _, f=f):
            idx16 = ids_s[pl.ds(g * 16, 16)]
            for j in range(16):
                pltpu.async_copy(ws[f].at[idx16[j]], stage_v.at[g * 16 + j],
                                 gsem)
            return 0

        lax.fori_loop(0, _BPW // 16, _fetch, 0)
        pltpu.make_async_copy(ws[f].at[pl.ds(0, _BPW)], stage_v, gsem).wait()

        pltpu.async_copy(stage_v, out_hbm.at[pl.ds(base, _BPW), f],
                         wsem).wait()


@jax.jit
def _run(xs, *ws):
    mesh = plsc.VectorSubcoreMesh(core_axis_name="c", subcore_axis_name="s")
    return pl.kernel(
        _body,
        out_type=jax.ShapeDtypeStruct((BATCH, NUM_FEATS, EMBED), jnp.float32),
        mesh=mesh,
        scratch_types=[
            pltpu.VMEM((_BPW,), jnp.int32),
            pltpu.VMEM((_BPW, EMBED), jnp.float32),
            pltpu.SemaphoreType.DMA,
            pltpu.SemaphoreType.DMA,
        ],
        compiler_params=pltpu.CompilerParams(needs_layout_passes=False),
    )(xs, *ws)


def kernel(X_0, X_1, X_2, X_3, X_4, X_5, X_6, X_7, X_8, X_9, X_10, X_11, X_12, X_13, X_14, X_15, X_16, X_17, X_18, X_19, X_20, X_21, X_22, X_23, X_24, X_25, W_0, W_1, W_2, W_3, W_4, W_5, W_6, W_7, W_8, W_9, W_10, W_11, W_12, W_13, W_14, W_15, W_16, W_17, W_18, W_19, W_20, W_21, W_22, W_23, W_24, W_25):
    xs = jnp.stack([X_0, X_1, X_2, X_3, X_4, X_5, X_6, X_7, X_8, X_9,
                    X_10, X_11, X_12, X_13, X_14, X_15, X_16, X_17, X_18,
                    X_19, X_20, X_21, X_22, X_23, X_24, X_25]).astype(jnp.int32)
    ws = (W_0, W_1, W_2, W_3, W_4, W_5, W_6, W_7, W_8, W_9, W_10, W_11,
          W_12, W_13, W_14, W_15, W_16, W_17, W_18, W_19, W_20, W_21,
          W_22, W_23, W_24, W_25)
    return _run(xs, *ws)


# speedup vs baseline: 1.4177x; 1.0317x over previous
import functools

import jax
import jax.numpy as jnp
from jax import lax
from jax.experimental import pallas as pl
from jax.experimental.pallas import tpu as pltpu
from jax.experimental.pallas import tpu_sc as plsc

NUM_FEATS = 26
VOCAB = 100000
EMBED = 64
BATCH = 4096

_NC = 2
_NS = 16
_NW = _NC * _NS
_BPW = BATCH // _NW
_NB = 4


def _body(xs_hbm, *refs):
    ws = refs[:NUM_FEATS]
    out_hbm = refs[NUM_FEATS]
    idx_v = refs[NUM_FEATS + 1]
    stages = refs[NUM_FEATS + 2:NUM_FEATS + 2 + _NB]
    gsems = refs[NUM_FEATS + 2 + _NB:NUM_FEATS + 2 + 2 * _NB]
    wsems = refs[NUM_FEATS + 2 + 2 * _NB:]

    wid = lax.axis_index("s") * _NC + lax.axis_index("c")
    base = wid * _BPW

    pltpu.sync_copy(xs_hbm.at[:, pl.ds(base, _BPW)], idx_v)

    def _enqueue(f, s):
        def _grp(g, _, f=f, s=s):
            idx16 = idx_v[f, pl.ds(g * 16, 16)]
            for j in range(16):
                pltpu.async_copy(ws[f].at[idx16[j]],
                                 stages[s].at[g * 16 + j], gsems[s])
            return 0
        lax.fori_loop(0, _BPW // 16, _grp, 0)

    def _drain_gather(f, s):
        pltpu.make_async_copy(ws[f].at[pl.ds(0, _BPW)], stages[s],
                              gsems[s]).wait()

    wd = [None] * NUM_FEATS
    for f in range(_NB):
        _enqueue(f, f)
    for f in range(NUM_FEATS):
        s = f % _NB
        _drain_gather(f, s)
        wd[f] = pltpu.async_copy(stages[s], out_hbm.at[pl.ds(base, _BPW), f],
                                 wsems[s])
        nf = f + _NB
        if nf < NUM_FEATS:
            wd[f].wait()
            _enqueue(nf, s)
    for f in range(NUM_FEATS - _NB, NUM_FEATS):
        wd[f].wait()


@jax.jit
def _run(xs, *ws):
    mesh = plsc.VectorSubcoreMesh(core_axis_name="c", subcore_axis_name="s")
    return pl.kernel(
        _body,
        out_type=jax.ShapeDtypeStruct((BATCH, NUM_FEATS, EMBED), jnp.float32),
        mesh=mesh,
        scratch_types=(
            [pltpu.VMEM((NUM_FEATS, _BPW), jnp.int32)]
            + [pltpu.VMEM((_BPW, EMBED), jnp.float32) for _ in range(_NB)]
            + [pltpu.SemaphoreType.DMA for _ in range(2 * _NB)]
        ),
        compiler_params=pltpu.CompilerParams(needs_layout_passes=False),
    )(xs, *ws)


def kernel(X_0, X_1, X_2, X_3, X_4, X_5, X_6, X_7, X_8, X_9, X_10, X_11, X_12, X_13, X_14, X_15, X_16, X_17, X_18, X_19, X_20, X_21, X_22, X_23, X_24, X_25, W_0, W_1, W_2, W_3, W_4, W_5, W_6, W_7, W_8, W_9, W_10, W_11, W_12, W_13, W_14, W_15, W_16, W_17, W_18, W_19, W_20, W_21, W_22, W_23, W_24, W_25):
    xs = jnp.stack([X_0, X_1, X_2, X_3, X_4, X_5, X_6, X_7, X_8, X_9,
                    X_10, X_11, X_12, X_13, X_14, X_15, X_16, X_17, X_18,
                    X_19, X_20, X_21, X_22, X_23, X_24, X_25]).astype(jnp.int32)
    ws = (W_0, W_1, W_2, W_3, W_4, W_5, W_6, W_7, W_8, W_9, W_10, W_11,
          W_12, W_13, W_14, W_15, W_16, W_17, W_18, W_19, W_20, W_21,
          W_22, W_23, W_24, W_25)
    return _run(xs, *ws)
